# Initial kernel scaffold; baseline (speedup 1.0000x reference)
import jax
import jax.numpy as jnp
from jax.experimental import pallas as pl


def kernel(input_ids, tok_embeddings, norm_weight):
    raise NotImplementedError("write your pallas kernel here")


# speedup vs baseline: 2.4358x; 2.4358x over previous
import functools

import jax
import jax.numpy as jnp
from jax import lax
from jax.experimental import pallas as pl
from jax.experimental.pallas import tpu as pltpu
from jax.experimental.pallas import tpu_sc as plsc

_EPS = 1e-6


def _lane_sum(x, L):
    iota = jnp.arange(L, dtype=jnp.int32)
    dnums = lax.GatherDimensionNumbers(
        offset_dims=(), collapsed_slice_dims=(0,), start_index_map=(0,)
    )
    for k in (1, 2, 4, 8):
        perm = jnp.asarray(iota ^ k, dtype=jnp.int32).reshape(L, 1)
        x = x + lax.gather(
            x,
            perm,
            dimension_numbers=dnums,
            slice_sizes=(1,),
            mode=lax.GatherScatterMode.PROMISE_IN_BOUNDS,
        )
    return x


def _vrsqrt(x):
    i = lax.bitcast_convert_type(x, jnp.int32)
    i = jnp.int32(0x5F3759DF) - lax.shift_right_arithmetic(i, 1)
    y = lax.bitcast_convert_type(i, jnp.float32)
    for _ in range(3):
        y = y * (1.5 - 0.5 * x * y * y)
    return y


@functools.cache
def _sc_geometry():
    info = plsc.get_sparse_core_info()
    return info.num_cores, info.num_subcores, info.num_lanes


@functools.cache
def _make_scale_table(V, D):
    NC, NS, L = _sc_geometry()
    NW = NC * NS
    assert V % NW == 0 and (V // NW) % 8 == 0 and D % L == 0
    rows_per_w = V // NW
    n_vreg = D // L
    mesh = plsc.VectorSubcoreMesh(core_axis_name="c", subcore_axis_name="s")

    @functools.partial(
        pl.kernel,
        mesh=mesh,
        out_type=jax.ShapeDtypeStruct((V, D), jnp.float32),
        scratch_types=[
            pltpu.VMEM((rows_per_w, D), jnp.float32),
            pltpu.VMEM((D,), jnp.float32),
        ],
    )
    def k(table_hbm, w_hbm, out_hbm, buf, wv):
        wid = lax.axis_index("s") * NC + lax.axis_index("c")
        base = wid * rows_per_w
        pltpu.sync_copy(w_hbm, wv)
        pltpu.sync_copy(table_hbm.at[pl.ds(base, rows_per_w)], buf)
        ws = [wv[pl.ds(j * L, L)] for j in range(n_vreg)]

        @plsc.parallel_loop(0, rows_per_w, unroll=2)
        def _(r):
            a0 = jnp.zeros((L,), jnp.float32)
            a1 = jnp.zeros((L,), jnp.float32)
            a2 = jnp.zeros((L,), jnp.float32)
            for j in range(0, n_vreg, 3):
                v = buf[r, pl.ds(j * L, L)]
                a0 = a0 + v * v
                v = buf[r, pl.ds((j + 1) * L, L)]
                a1 = a1 + v * v
                v = buf[r, pl.ds((j + 2) * L, L)]
                a2 = a2 + v * v
            s = _lane_sum(a0 + a1 + a2, L) * (1.0 / D) + _EPS
            inv = _vrsqrt(s)
            for j in range(n_vreg):
                buf[r, pl.ds(j * L, L)] = buf[r, pl.ds(j * L, L)] * (ws[j] * inv)

        pltpu.sync_copy(buf, out_hbm.at[pl.ds(base, rows_per_w)])

    return k


@functools.cache
def _make_gather(V, D, B):
    NC, NS, L = _sc_geometry()
    NW = NC * NS
    assert B % (8 * NW) == 0 and D % L == 0
    b_per_w = B // NW
    C = 128
    n_chunks = b_per_w // C
    mesh = plsc.VectorSubcoreMesh(core_axis_name="c", subcore_axis_name="s")

    @functools.partial(
        pl.kernel,
        mesh=mesh,
        out_type=jax.ShapeDtypeStruct((B, D), jnp.float32),
        scratch_types=[
            pltpu.VMEM((b_per_w,), jnp.int32),
            pltpu.VMEM((C, D), jnp.float32),
            pltpu.VMEM((C, D), jnp.float32),
            pltpu.SemaphoreType.DMA,
            pltpu.SemaphoreType.DMA,
            pltpu.SemaphoreType.DMA,
            pltpu.SemaphoreType.DMA,
        ],
    )
    def k(ids_hbm, st_hbm, out_hbm, idx_v, buf0, buf1, g0, g1, s0, s1):
        wid = lax.axis_index("s") * NC + lax.axis_index("c")
        base = wid * b_per_w
        pltpu.sync_copy(ids_hbm.at[pl.ds(base, b_per_w)], idx_v)
        bufs = (buf0, buf1)
        gsem = (g0, g1)
        ssem = (s0, s1)

        def gather(c):
            return pltpu.async_copy(
                st_hbm.at[idx_v.at[pl.ds(c * C, C)]], bufs[c % 2], gsem[c % 2]
            )

        def store(c):
            return pltpu.async_copy(
                bufs[c % 2], out_hbm.at[pl.ds(base + c * C, C)], ssem[c % 2]
            )

        gathers = [gather(0), gather(1)]
        stores = [None, None]
        for c in range(n_chunks):
            gathers[c % 2].wait()
            stores[c % 2] = store(c)
            if c + 2 < n_chunks:
                stores[c % 2].wait()
                gathers[c % 2] = gather(c + 2)
        stores[(n_chunks - 2) % 2].wait()
        stores[(n_chunks - 1) % 2].wait()

    return k


def kernel(input_ids, tok_embeddings, norm_weight):
    Bt, S = input_ids.shape
    V, D = tok_embeddings.shape
    scaled = _make_scale_table(V, D)(tok_embeddings, norm_weight)
    ids = input_ids.reshape(-1)
    out = _make_gather(V, D, Bt * S)(ids, scaled)
    return out.reshape(Bt, S, D)
